# baseline (speedup 1.0000x reference)
import functools

import jax
import jax.numpy as jnp
from jax import lax
from jax.experimental import pallas as pl
from jax.experimental.pallas import tpu as pltpu
from jax.experimental.pallas import tpu_sc as plsc

T, D, E, K, H = 2048, 768, 8, 2, 1536
TM = 256
NP = T * K
S = NP + E * TM
NT = S // TM
NC, NS, L = 2, 16, 16
NW = NC * NS
CH = NP // NW
SLOTS_W = S // NW
TOK_W = T // NW

_mesh = plsc.VectorSubcoreMesh(core_axis_name="c", subcore_axis_name="s")
_sc_params = pltpu.CompilerParams(needs_layout_passes=False)


def _wid():
    return lax.axis_index("s") * NC + lax.axis_index("c")


def _route_body(ef_hbm, gf_hbm,
                tok_hbm, gfull_hbm, dest_hbm, meta_hbm, stats_hbm,
                ef_v, gf_v, dest2_v, tok2_v, gate2_v, pidx_v, pval_v,
                misc_v, stage_v, tbl_c_v, tbl_g_v,
                cnt_sh, gsum_sh, sem0, sem1):
    cid = lax.axis_index("c")
    sid = lax.axis_index("s")
    wid = sid * NC + cid
    lane = lax.iota(jnp.int32, 16)
    zeros_i = jnp.zeros((16,), jnp.int32)
    zeros_f = jnp.zeros((16,), jnp.float32)

    pltpu.sync_copy(ef_hbm.at[pl.ds(sid * 2 * CH, 2 * CH)], ef_v)
    pltpu.sync_copy(gf_hbm.at[pl.ds(sid * 2 * CH, 2 * CH)], gf_v)

    cnt_row = zeros_i
    gsum_row = zeros_f
    for cc in range(2):
        acc_c = [zeros_i for _ in range(E)]
        acc_g = [zeros_f for _ in range(E)]
        for j in range(CH // 16):
            v = ef_v[pl.ds(cc * CH + j * 16, 16)]
            g = gf_v[pl.ds(cc * CH + j * 16, 16)]
            for e in range(E):
                m = v == e
                acc_c[e] = acc_c[e] + jnp.where(m, 1, 0)
                acc_g[e] = acc_g[e] + jnp.where(m, g, 0.0)
        for e in range(E):
            cnt_row = jnp.where(lane == cc * E + e, jnp.sum(acc_c[e]), cnt_row)
            gsum_row = jnp.where(lane == cc * E + e, jnp.sum(acc_g[e]),
                                 gsum_row)
    stage_v[pl.ds(0, 16)] = cnt_row
    pltpu.sync_copy(stage_v.at[pl.ds(0, 16)], cnt_sh.at[pl.ds(sid * 16, 16)])
    gate2_v[0, pl.ds(0, 16)] = gsum_row
    pltpu.sync_copy(gate2_v.at[0, pl.ds(0, 16)],
                    gsum_sh.at[pl.ds(sid * 16, 16)])
    with jax.named_scope("ph_barrier"):
        plsc.subcore_barrier()
    pltpu.sync_copy(cnt_sh, tbl_c_v)
    pltpu.sync_copy(gsum_sh, tbl_g_v)

    tot16_c = zeros_i
    tot16_g = zeros_f
    pre16 = zeros_i
    for s in range(NS):
        r_c = tbl_c_v[pl.ds(s * 16, 16)]
        r_g = tbl_g_v[pl.ds(s * 16, 16)]
        tot16_c = tot16_c + r_c
        tot16_g = tot16_g + r_g
        m_even = jnp.where(2 * s < wid, 1, 0)
        m_odd = jnp.where(2 * s + 1 < wid, 1, 0)
        pre16 = pre16 + r_c * jnp.where(lane < E, m_even, m_odd)
    counts_vec = zeros_i
    gsum_vec = zeros_f
    pre_vec = zeros_i
    for e in range(E):
        m2 = (lane == e) | (lane == E + e)
        counts_vec = jnp.where(lane == e, jnp.sum(jnp.where(m2, tot16_c, 0)),
                               counts_vec)
        gsum_vec = jnp.where(lane == e, jnp.sum(jnp.where(m2, tot16_g, 0.0)),
                             gsum_vec)
        pre_vec = jnp.where(lane == e, jnp.sum(jnp.where(m2, pre16, 0)),
                            pre_vec)

    padded_vec = (counts_vec + (TM - 1)) & (-TM)
    cum_incl = plsc.cumsum(padded_vec)
    offs_vec = cum_incl - padded_vec
    used = jnp.sum(jnp.where(lane == E - 1, cum_incl, 0))

    base = []
    for e in range(E):
        b_e = jnp.sum(jnp.where(lane == e, offs_vec + pre_vec, 0))
        base.append(zeros_i + b_e)
    start = wid * CH
    kk = jnp.where(start >= T, T, 0)
    for j in range(CH // 16):
        v = ef_v[pl.ds(cid * CH + j * 16, 16)]
        g = gf_v[pl.ds(cid * CH + j * 16, 16)]
        destv = zeros_i
        for e in range(E):
            m = v == e
            csum = plsc.cumsum(jnp.where(m, 1, 0))
            destv = jnp.where(m, base[e] + csum - 1, destv)
            base[e] = base[e] + plsc.all_reduce_population_count(m)
        b, col = j // 4, (j % 4) * 16
        dest2_v[b, pl.ds(col, 16)] = destv
        tok2_v[b, pl.ds(col, 16)] = (start + j * 16 - kk) + lane
        gate2_v[b, pl.ds(col, 16)] = g

    with jax.named_scope("ph_destdma"):
        pltpu.sync_copy(dest2_v.at[0],
                        dest_hbm.at[pl.ds(start, CH // 2)])
    pltpu.sync_copy(dest2_v.at[1],
                    dest_hbm.at[pl.ds(start + CH // 2, CH // 2)])
    pltpu.async_copy(tok2_v.at[0], tok_hbm.at[dest2_v.at[0]], sem0).wait()
    pltpu.async_copy(tok2_v.at[1], tok_hbm.at[dest2_v.at[1]], sem1).wait()
    pltpu.async_copy(gate2_v.at[0], gfull_hbm.at[dest2_v.at[0]], sem0).wait()
    pltpu.async_copy(gate2_v.at[1], gfull_hbm.at[dest2_v.at[1]], sem1).wait()

    @pl.when(wid < E)
    def _pad():
        cnt_e = jnp.sum(jnp.where(lane == wid, counts_vec, 0))
        off_e = jnp.sum(jnp.where(lane == wid, offs_vec, 0))
        pad_e = jnp.sum(jnp.where(lane == wid, padded_vec, 0))
        pad_start = off_e + cnt_e
        pad_len = pad_e - cnt_e
        for b in range(2):
            for j in range(CH // 16):
                o = b * CH + j * 16 + lane
                pidx_v[pl.ds(j * 16, 16)] = jnp.where(
                    o < pad_len, pad_start + o, S + wid * (2 * CH) + o)
                pval_v[pl.ds(j * 16, 16)] = (pad_start + o) & (T - 1)
            pltpu.async_copy(pval_v, tok_hbm.at[pidx_v], sem0).wait()

    @pl.when(wid == E)
    def _tail():
        for r in range((S - NP) // CH):
            tstart = NP + r * CH

            @pl.when(tstart >= used)
            def _z():
                for j in range(CH // 16):
                    pval_v[pl.ds(j * 16, 16)] = (tstart + j * 16 + lane) & (
                        T - 1)
                pltpu.sync_copy(pval_v, tok_hbm.at[pl.ds(tstart, CH)])

    @pl.when(wid == E + 1)
    def _meta():
      with jax.named_scope("ph_meta"):
        used_tiles = zeros_i + jnp.sum(jnp.where(lane == 0, used, 0) >> 8)
        last_e = zeros_i
        for e in range(E):
            cum_e = jnp.sum(jnp.where(lane == e, cum_incl, 0))
            last_e = last_e + jnp.where((zeros_i + used) - TM >= cum_e, 1, 0)
        for h in range(2):
            pos = (h * 16 + lane) * TM
            te = zeros_i
            for e in range(E):
                cum_e = jnp.sum(jnp.where(lane == e, cum_incl, 0))
                te = te + jnp.where(pos >= cum_e, 1, 0)
            te = jnp.minimum(te, last_e)
            if h == 1:
                te = jnp.where(lane < NT - 16, te, used_tiles)
            misc_v[pl.ds(h * 16, 16)] = te
        pltpu.sync_copy(misc_v, meta_hbm)

    @pl.when(wid == E + 2)
    def _stats():
      with jax.named_scope("ph_stats"):
        m8 = lane < E

        def cv2(vec):
            mean = zeros_f + jnp.sum(jnp.where(m8, vec, 0.0)) * (1.0 / E)
            d = jnp.where(m8, vec - mean, 0.0)
            var = zeros_f + jnp.sum(d * d) * (1.0 / (E - 1))
            return var / (mean * mean + 1e-10)

        loss = cv2(gsum_vec) + cv2(counts_vec.astype(jnp.float32))
        misc_v[pl.ds(0, 16)] = zeros_i
        gate2_v[0, pl.ds(0, 16)] = jnp.where(lane == 0, loss, 0.0)
        pltpu.sync_copy(gate2_v.at[0, pl.ds(0, 16)], stats_hbm)


_route = pl.kernel(
    _route_body,
    out_type=[jax.ShapeDtypeStruct((S + NW * 2 * CH,), jnp.int32),
              jax.ShapeDtypeStruct((S,), jnp.float32),
              jax.ShapeDtypeStruct((NP,), jnp.int32),
              jax.ShapeDtypeStruct((32,), jnp.int32),
              jax.ShapeDtypeStruct((16,), jnp.float32)],
    mesh=_mesh,
    scratch_types=[pltpu.VMEM((2 * CH,), jnp.int32),
                   pltpu.VMEM((2 * CH,), jnp.float32),
                   pltpu.VMEM((2, CH // 2), jnp.int32),
                   pltpu.VMEM((2, CH // 2), jnp.int32),
                   pltpu.VMEM((2, CH // 2), jnp.float32),
                   pltpu.VMEM((CH,), jnp.int32),
                   pltpu.VMEM((CH,), jnp.int32),
                   pltpu.VMEM((32,), jnp.int32),
                   pltpu.VMEM((16,), jnp.int32),
                   pltpu.VMEM((NS * 16,), jnp.int32),
                   pltpu.VMEM((NS * 16,), jnp.float32),
                   pltpu.VMEM_SHARED((NS * 16,), jnp.int32),
                   pltpu.VMEM_SHARED((NS * 16,), jnp.float32),
                   pltpu.SemaphoreType.DMA,
                   pltpu.SemaphoreType.DMA],
    compiler_params=_sc_params,
)


GCH = 64


def _gather_body(tok3d_hbm, x_hbm, xs_hbm, idx_v, rows0_v, rows1_v, sem0,
                 sem1):
    wid = _wid()
    nch = SLOTS_W // GCH
    pltpu.sync_copy(tok3d_hbm.at[wid], idx_v)
    bufs = [rows0_v, rows1_v]
    sems = [sem0, sem1]
    cps = [None] * nch
    cps[0] = pltpu.async_copy(x_hbm.at[idx_v.at[0]], bufs[0], sems[0])
    for c in range(nch):
        cps[c].wait()
        if c + 1 < nch:
            cps[c + 1] = pltpu.async_copy(x_hbm.at[idx_v.at[c + 1]],
                                          bufs[(c + 1) % 2], sems[(c + 1) % 2])
        pltpu.sync_copy(bufs[c % 2],
                        xs_hbm.at[pl.ds(wid * SLOTS_W + c * GCH, GCH)])


_gatherx = pl.kernel(
    _gather_body,
    out_type=jax.ShapeDtypeStruct((S, D), jnp.float32),
    mesh=_mesh,
    scratch_types=[pltpu.VMEM((SLOTS_W // GCH, GCH), jnp.int32),
                   pltpu.VMEM((GCH, D), jnp.float32),
                   pltpu.VMEM((GCH, D), jnp.float32),
                   pltpu.SemaphoreType.DMA,
                   pltpu.SemaphoreType.DMA],
    compiler_params=_sc_params,
)


def _mlp_body(meta_ref, xs_ref, w1_ref, b1_ref, w2_ref, b2_ref, gate_ref,
              ys_ref):
    @pl.when(pl.program_id(0) < meta_ref[NT])
    def _compute():
        xb = xs_ref[...]
        h = jnp.dot(xb, w1_ref[0], preferred_element_type=jnp.float32)
        h = jnp.maximum(h + b1_ref[0, 0][None, :], 0.0)
        o = jnp.dot(h, w2_ref[0], preferred_element_type=jnp.float32)
        o = o + b2_ref[0, 0][None, :]
        ys_ref[...] = o * gate_ref[0, 0][:, None]


def _run_mlp(meta, xs, W1, b1, W2, b2, gate3d):
    grid_spec = pltpu.PrefetchScalarGridSpec(
        num_scalar_prefetch=1,
        grid=(NT,),
        in_specs=[
            pl.BlockSpec((TM, D), lambda nt, m: (nt, 0)),
            pl.BlockSpec((1, D, H), lambda nt, m: (m[nt], 0, 0)),
            pl.BlockSpec((1, 1, H), lambda nt, m: (m[nt], 0, 0)),
            pl.BlockSpec((1, H, D), lambda nt, m: (m[nt], 0, 0)),
            pl.BlockSpec((1, 1, D), lambda nt, m: (m[nt], 0, 0)),
            pl.BlockSpec((1, 1, TM), lambda nt, m: (nt, 0, 0)),
        ],
        out_specs=pl.BlockSpec((TM, D), lambda nt, m: (nt, 0)),
    )
    return pl.pallas_call(
        _mlp_body,
        grid_spec=grid_spec,
        out_shape=jax.ShapeDtypeStruct((S, D), jnp.float32),
    )(meta, xs, W1, b1, W2, b2, gate3d)


CCH = 32


def _combine_body(ys_hbm, dest3d_hbm, y_hbm, idx_v, a0_v, b0_v, a1_v, b1_v,
                  sem0, sem1):
    wid = _wid()
    nch = TOK_W // CCH
    pltpu.sync_copy(dest3d_hbm.at[wid], idx_v.at[pl.ds(0, 1)])
    pltpu.sync_copy(dest3d_hbm.at[NW + wid], idx_v.at[pl.ds(1, 1)])
    a_bufs, b_bufs, sems = [a0_v, a1_v], [b0_v, b1_v], [sem0, sem1]

    def fire(c):
        p = c % 2
        cpa = pltpu.async_copy(ys_hbm.at[idx_v.at[0, pl.ds(c * CCH, CCH)]],
                               a_bufs[p], sems[p])
        cpb = pltpu.async_copy(ys_hbm.at[idx_v.at[1, pl.ds(c * CCH, CCH)]],
                               b_bufs[p], sems[p])
        return cpa, cpb

    cps = [None] * nch
    cps[0] = fire(0)
    for c in range(nch):
        p = c % 2
        cps[c][0].wait()
        cps[c][1].wait()
        if c + 1 < nch:
            cps[c + 1] = fire(c + 1)
        a_v, b_v = a_bufs[p], b_bufs[p]

        def row_body(r, _):
            for q in range(D // 16):
                a_v[r, pl.ds(q * 16, 16)] = (a_v[r, pl.ds(q * 16, 16)] +
                                             b_v[r, pl.ds(q * 16, 16)])
            return 0

        lax.fori_loop(0, CCH, row_body, 0)
        pltpu.sync_copy(a_v, y_hbm.at[pl.ds(wid * TOK_W + c * CCH, CCH)])


_combine = pl.kernel(
    _combine_body,
    out_type=jax.ShapeDtypeStruct((T, D), jnp.float32),
    mesh=_mesh,
    scratch_types=[pltpu.VMEM((2, TOK_W), jnp.int32),
                   pltpu.VMEM((CCH, D), jnp.float32),
                   pltpu.VMEM((CCH, D), jnp.float32),
                   pltpu.VMEM((CCH, D), jnp.float32),
                   pltpu.VMEM((CCH, D), jnp.float32),
                   pltpu.SemaphoreType.DMA,
                   pltpu.SemaphoreType.DMA],
    compiler_params=_sc_params,
)


def kernel(x, w_gate, W1, b1, W2, b2):
    logits = x @ w_gate
    i1 = jnp.argmax(logits, axis=-1)
    oh1 = i1[:, None] == jnp.arange(E)[None, :]
    v1 = jnp.max(logits, axis=-1)
    l2 = jnp.where(oh1, -jnp.inf, logits)
    i2 = jnp.argmax(l2, axis=-1)
    v2 = jnp.max(l2, axis=-1)
    top_gates = jax.nn.softmax(jnp.stack([v1, v2], axis=-1), axis=-1)
    ef = jnp.concatenate([i1, i2]).astype(jnp.int32)
    gf = top_gates.T.reshape(-1)

    tok, gfull, dest, meta, stats = _route(ef, gf)
    xs = _gatherx(tok[:S].reshape(NW, SLOTS_W // GCH, GCH), x)
    ys = _run_mlp(meta, xs, W1, b1.reshape(E, 1, H), W2, b2.reshape(E, 1, D),
                  gfull.reshape(NT, 1, TM))
    y = _combine(ys, dest.reshape(K * NW, 1, TOK_W))
    return y, stats[0]


# speedup vs baseline: 1.2570x; 1.2570x over previous
import jax
import jax.numpy as jnp
from jax import lax
from jax.experimental import pallas as pl
from jax.experimental.pallas import tpu as pltpu
from jax.experimental.pallas import tpu_sc as plsc

T, D, E, K, H = 2048, 768, 8, 2, 1536
TM = 256
NP = T * K
S = NP + E * TM
NT = S // TM
NC, NS, L = 2, 16, 16
NW = NC * NS
CH = NP // NW
TOK_W = T // NW

_mesh = plsc.VectorSubcoreMesh(core_axis_name="c", subcore_axis_name="s")
_sc_params = pltpu.CompilerParams(needs_layout_passes=False)


def _wid():
    return lax.axis_index("s") * NC + lax.axis_index("c")


def _route_body(ef_hbm, gf_hbm, x_hbm,
                xs_hbm, gfull_hbm, dest_hbm, meta_hbm, stats_hbm,
                ef_v, gf_v, dest2_v, tok2_v, gate2_v,
                misc_v, rows0_v, rows1_v, stage_v, tbl_c_v, tbl_g_v,
                cnt_sh, gsum_sh, sem0, sem1):
    cid = lax.axis_index("c")
    sid = lax.axis_index("s")
    wid = sid * NC + cid
    lane = lax.iota(jnp.int32, 16)
    zeros_i = jnp.zeros((16,), jnp.int32)
    zeros_f = jnp.zeros((16,), jnp.float32)

    pltpu.sync_copy(ef_hbm.at[pl.ds(sid * 2 * CH, 2 * CH)], ef_v)
    pltpu.sync_copy(gf_hbm.at[pl.ds(sid * 2 * CH, 2 * CH)], gf_v)

    cnt_row = zeros_i
    gsum_row = zeros_f
    for cc in range(2):
        acc_c = [zeros_i for _ in range(E)]
        acc_g = [zeros_f for _ in range(E)]
        for j in range(CH // 16):
            v = ef_v[pl.ds(cc * CH + j * 16, 16)]
            g = gf_v[pl.ds(cc * CH + j * 16, 16)]
            for e in range(E):
                m = v == e
                acc_c[e] = acc_c[e] + jnp.where(m, 1, 0)
                acc_g[e] = acc_g[e] + jnp.where(m, g, 0.0)
        for e in range(E):
            cnt_row = jnp.where(lane == cc * E + e, jnp.sum(acc_c[e]), cnt_row)
            gsum_row = jnp.where(lane == cc * E + e, jnp.sum(acc_g[e]),
                                 gsum_row)
    stage_v[pl.ds(0, 16)] = cnt_row
    pltpu.sync_copy(stage_v.at[pl.ds(0, 16)], cnt_sh.at[pl.ds(sid * 16, 16)])
    gate2_v[0, pl.ds(0, 16)] = gsum_row
    pltpu.sync_copy(gate2_v.at[0, pl.ds(0, 16)],
                    gsum_sh.at[pl.ds(sid * 16, 16)])
    plsc.subcore_barrier()
    pltpu.sync_copy(cnt_sh, tbl_c_v)
    pltpu.sync_copy(gsum_sh, tbl_g_v)

    tot16_c = zeros_i
    tot16_g = zeros_f
    pre16 = zeros_i
    for s in range(NS):
        r_c = tbl_c_v[pl.ds(s * 16, 16)]
        r_g = tbl_g_v[pl.ds(s * 16, 16)]
        tot16_c = tot16_c + r_c
        tot16_g = tot16_g + r_g
        m_even = jnp.where(2 * s < wid, 1, 0)
        m_odd = jnp.where(2 * s + 1 < wid, 1, 0)
        pre16 = pre16 + r_c * jnp.where(lane < E, m_even, m_odd)
    counts_vec = zeros_i
    gsum_vec = zeros_f
    pre_vec = zeros_i
    for e in range(E):
        m2 = (lane == e) | (lane == E + e)
        counts_vec = jnp.where(lane == e, jnp.sum(jnp.where(m2, tot16_c, 0)),
                               counts_vec)
        gsum_vec = jnp.where(lane == e, jnp.sum(jnp.where(m2, tot16_g, 0.0)),
                             gsum_vec)
        pre_vec = jnp.where(lane == e, jnp.sum(jnp.where(m2, pre16, 0)),
                            pre_vec)

    padded_vec = (counts_vec + (TM - 1)) & (-TM)
    cum_incl = plsc.cumsum(padded_vec)
    offs_vec = cum_incl - padded_vec
    used = jnp.sum(jnp.where(lane == E - 1, cum_incl, 0))

    base = []
    for e in range(E):
        b_e = jnp.sum(jnp.where(lane == e, offs_vec + pre_vec, 0))
        base.append(zeros_i + b_e)
    start = wid * CH
    kk = jnp.where(start >= T, T, 0)
    for j in range(CH // 16):
        v = ef_v[pl.ds(cid * CH + j * 16, 16)]
        g = gf_v[pl.ds(cid * CH + j * 16, 16)]
        destv = zeros_i
        for e in range(E):
            m = v == e
            csum = plsc.cumsum(jnp.where(m, 1, 0))
            destv = jnp.where(m, base[e] + csum - 1, destv)
            base[e] = base[e] + plsc.all_reduce_population_count(m)
        b, col = j // 4, (j % 4) * 16
        dest2_v[b, pl.ds(col, 16)] = destv
        tok2_v[b, pl.ds(col, 16)] = (start + j * 16 - kk) + lane
        gate2_v[b, pl.ds(col, 16)] = g

    pltpu.sync_copy(dest2_v.at[0], dest_hbm.at[pl.ds(start, CH // 2)])
    pltpu.sync_copy(dest2_v.at[1],
                    dest_hbm.at[pl.ds(start + CH // 2, CH // 2)])
    cpg0 = pltpu.async_copy(x_hbm.at[tok2_v.at[0]], rows0_v, sem0)
    cpg1 = pltpu.async_copy(x_hbm.at[tok2_v.at[1]], rows1_v, sem1)
    cpg0.wait()
    cps0 = pltpu.async_copy(rows0_v, xs_hbm.at[dest2_v.at[0]], sem0)
    cpg1.wait()
    cps1 = pltpu.async_copy(rows1_v, xs_hbm.at[dest2_v.at[1]], sem1)
    cps0.wait()
    cps1.wait()
    pltpu.async_copy(gate2_v.at[0], gfull_hbm.at[dest2_v.at[0]], sem0).wait()
    pltpu.async_copy(gate2_v.at[1], gfull_hbm.at[dest2_v.at[1]], sem1).wait()

    @pl.when(wid == E + 1)
    def _meta():
        used_tiles = zeros_i + jnp.sum(jnp.where(lane == 0, used, 0) >> 8)
        last_e = zeros_i
        for e in range(E):
            cum_e = jnp.sum(jnp.where(lane == e, cum_incl, 0))
            last_e = last_e + jnp.where((zeros_i + used) - TM >= cum_e, 1, 0)
        for h in range(2):
            pos = (h * 16 + lane) * TM
            te = zeros_i
            for e in range(E):
                cum_e = jnp.sum(jnp.where(lane == e, cum_incl, 0))
                te = te + jnp.where(pos >= cum_e, 1, 0)
            te = jnp.minimum(te, last_e)
            if h == 1:
                te = jnp.where(lane < NT - 16, te, used_tiles)
            misc_v[pl.ds(h * 16, 16)] = te
        pltpu.sync_copy(misc_v, meta_hbm)

    @pl.when(wid == E + 2)
    def _stats():
        m8 = lane < E

        def cv2(vec):
            mean = zeros_f + jnp.sum(jnp.where(m8, vec, 0.0)) * (1.0 / E)
            d = jnp.where(m8, vec - mean, 0.0)
            var = zeros_f + jnp.sum(d * d) * (1.0 / (E - 1))
            return var / (mean * mean + 1e-10)

        loss = cv2(gsum_vec) + cv2(counts_vec.astype(jnp.float32))
        misc_v[pl.ds(0, 16)] = zeros_i
        gate2_v[0, pl.ds(0, 16)] = jnp.where(lane == 0, loss, 0.0)
        pltpu.sync_copy(gate2_v.at[0, pl.ds(0, 16)], stats_hbm)


_route = pl.kernel(
    _route_body,
    out_type=[jax.ShapeDtypeStruct((S, D), jnp.float32),
              jax.ShapeDtypeStruct((S,), jnp.float32),
              jax.ShapeDtypeStruct((NP,), jnp.int32),
              jax.ShapeDtypeStruct((32,), jnp.int32),
              jax.ShapeDtypeStruct((16,), jnp.float32)],
    mesh=_mesh,
    scratch_types=[pltpu.VMEM((2 * CH,), jnp.int32),
                   pltpu.VMEM((2 * CH,), jnp.float32),
                   pltpu.VMEM((2, CH // 2), jnp.int32),
                   pltpu.VMEM((2, CH // 2), jnp.int32),
                   pltpu.VMEM((2, CH // 2), jnp.float32),
                   pltpu.VMEM((32,), jnp.int32),
                   pltpu.VMEM((CH // 2, D), jnp.float32),
                   pltpu.VMEM((CH // 2, D), jnp.float32),
                   pltpu.VMEM((16,), jnp.int32),
                   pltpu.VMEM((NS * 16,), jnp.int32),
                   pltpu.VMEM((NS * 16,), jnp.float32),
                   pltpu.VMEM_SHARED((NS * 16,), jnp.int32),
                   pltpu.VMEM_SHARED((NS * 16,), jnp.float32),
                   pltpu.SemaphoreType.DMA,
                   pltpu.SemaphoreType.DMA],
    compiler_params=_sc_params,
)


def _mlp_body(meta_ref, xs_ref, w1_ref, b1_ref, w2_ref, b2_ref, gate_ref,
              ys_ref):
    @pl.when(pl.program_id(0) < meta_ref[NT])
    def _compute():
        xb = xs_ref[...]
        h = jnp.dot(xb, w1_ref[0], preferred_element_type=jnp.float32)
        h = jnp.maximum(h + b1_ref[0, 0][None, :], 0.0)
        o = jnp.dot(h, w2_ref[0], preferred_element_type=jnp.float32)
        o = o + b2_ref[0, 0][None, :]
        ys_ref[...] = o * gate_ref[0, 0][:, None]


def _run_mlp(meta, xs, W1, b1, W2, b2, gate3d):
    grid_spec = pltpu.PrefetchScalarGridSpec(
        num_scalar_prefetch=1,
        grid=(NT,),
        in_specs=[
            pl.BlockSpec((TM, D), lambda nt, m: (nt, 0)),
            pl.BlockSpec((1, D, H), lambda nt, m: (m[nt], 0, 0)),
            pl.BlockSpec((1, 1, H), lambda nt, m: (m[nt], 0, 0)),
            pl.BlockSpec((1, H, D), lambda nt, m: (m[nt], 0, 0)),
            pl.BlockSpec((1, 1, D), lambda nt, m: (m[nt], 0, 0)),
            pl.BlockSpec((1, 1, TM), lambda nt, m: (nt, 0, 0)),
        ],
        out_specs=pl.BlockSpec((TM, D), lambda nt, m: (nt, 0)),
    )
    return pl.pallas_call(
        _mlp_body,
        grid_spec=grid_spec,
        out_shape=jax.ShapeDtypeStruct((S, D), jnp.float32),
    )(meta, xs, W1, b1, W2, b2, gate3d)


CCH = 32


def _combine_body(ys_hbm, dest3d_hbm, y_hbm, idx_v, a0_v, b0_v, a1_v, b1_v,
                  sem0, sem1):
    wid = _wid()
    nch = TOK_W // CCH
    pltpu.sync_copy(dest3d_hbm.at[wid], idx_v.at[pl.ds(0, 1)])
    pltpu.sync_copy(dest3d_hbm.at[NW + wid], idx_v.at[pl.ds(1, 1)])
    a_bufs, b_bufs, sems = [a0_v, a1_v], [b0_v, b1_v], [sem0, sem1]

    def fire(c):
        p = c % 2
        cpa = pltpu.async_copy(ys_hbm.at[idx_v.at[0, pl.ds(c * CCH, CCH)]],
                               a_bufs[p], sems[p])
        cpb = pltpu.async_copy(ys_hbm.at[idx_v.at[1, pl.ds(c * CCH, CCH)]],
                               b_bufs[p], sems[p])
        return cpa, cpb

    cps = [None] * nch
    cps[0] = fire(0)
    for c in range(nch):
        p = c % 2
        cps[c][0].wait()
        cps[c][1].wait()
        if c + 1 < nch:
            cps[c + 1] = fire(c + 1)
        a_v, b_v = a_bufs[p], b_bufs[p]

        def row_body(r, _):
            for q in range(D // 16):
                a_v[r, pl.ds(q * 16, 16)] = (a_v[r, pl.ds(q * 16, 16)] +
                                             b_v[r, pl.ds(q * 16, 16)])
            return 0

        lax.fori_loop(0, CCH, row_body, 0)
        pltpu.sync_copy(a_v, y_hbm.at[pl.ds(wid * TOK_W + c * CCH, CCH)])


_combine = pl.kernel(
    _combine_body,
    out_type=jax.ShapeDtypeStruct((T, D), jnp.float32),
    mesh=_mesh,
    scratch_types=[pltpu.VMEM((2, TOK_W), jnp.int32),
                   pltpu.VMEM((CCH, D), jnp.float32),
                   pltpu.VMEM((CCH, D), jnp.float32),
                   pltpu.VMEM((CCH, D), jnp.float32),
                   pltpu.VMEM((CCH, D), jnp.float32),
                   pltpu.SemaphoreType.DMA,
                   pltpu.SemaphoreType.DMA],
    compiler_params=_sc_params,
)


def kernel(x, w_gate, W1, b1, W2, b2):
    logits = x @ w_gate
    i1 = jnp.argmax(logits, axis=-1)
    oh1 = i1[:, None] == jnp.arange(E)[None, :]
    v1 = jnp.max(logits, axis=-1)
    l2 = jnp.where(oh1, -jnp.inf, logits)
    i2 = jnp.argmax(l2, axis=-1)
    v2 = jnp.max(l2, axis=-1)
    top_gates = jax.nn.softmax(jnp.stack([v1, v2], axis=-1), axis=-1)
    ef = jnp.concatenate([i1, i2]).astype(jnp.int32)
    gf = top_gates.T.reshape(-1)

    xs, gfull, dest, meta, stats = _route(ef, gf, x)
    ys = _run_mlp(meta, xs, W1, b1.reshape(E, 1, H), W2, b2.reshape(E, 1, D),
                  gfull.reshape(NT, 1, TM))
    y = _combine(ys, dest.reshape(K * NW, 1, TOK_W))
    return y, stats[0]
